# baseline (speedup 1.0000x reference)
import functools

import jax
import jax.numpy as jnp
from jax import lax
from jax.experimental import pallas as pl
from jax.experimental.pallas import tpu as pltpu
from jax.experimental.pallas import tpu_sc as plsc

B = 4096
N_TOK = 200
D = 64
NC = 2
NS = 16
NW = NC * NS
ROWS_PER_W = B // NW
LANES = 16
IDX_MINOR = 100
IDX_MAJOR = N_TOK // IDX_MINOR

_mesh = plsc.VectorSubcoreMesh(core_axis_name="c", subcore_axis_name="s")


@functools.partial(
    pl.kernel,
    mesh=_mesh,
    compiler_params=pltpu.CompilerParams(use_tc_tiling_on_sc=False),
    out_type=jax.ShapeDtypeStruct((B, N_TOK, D), jnp.float32),
    scratch_types=[
        pltpu.VMEM((IDX_MAJOR, IDX_MINOR), jnp.int32),
        pltpu.VMEM((N_TOK, D), jnp.float32),
        pltpu.VMEM((N_TOK, D), jnp.float32),
        pltpu.SemaphoreType.DMA,
    ],
)
def _emb_kernel(x_hbm, table_hbm, pos_hbm, out_hbm, idx_v, rows_v, pos_v, sem):
    wid = lax.axis_index("s") * NC + lax.axis_index("c")
    base = wid * ROWS_PER_W
    pltpu.sync_copy(pos_hbm, pos_v)

    def row_body(i, carry):
        b = base + i
        pltpu.sync_copy(x_hbm.at[b], idx_v)
        cps = []
        for j in range(IDX_MAJOR):
            cps.append(
                pltpu.async_copy(
                    table_hbm.at[idx_v.at[j]],
                    rows_v.at[pl.ds(j * IDX_MINOR, IDX_MINOR)],
                    sem,
                )
            )
        for cp in cps:
            cp.wait()

        def add_body(r, c):
            for k in range(D // LANES):
                sl = pl.ds(k * LANES, LANES)
                rows_v[r, sl] = rows_v[r, sl] + pos_v[r, sl]
            return c

        lax.fori_loop(0, N_TOK, add_body, 0)
        pltpu.sync_copy(rows_v, out_hbm.at[b])
        return carry

    lax.fori_loop(0, ROWS_PER_W, row_body, 0)


def kernel(x, table, pos_embedding):
    x3 = x.reshape(B, IDX_MAJOR, IDX_MINOR).astype(jnp.int32)
    return _emb_kernel(x3, table, pos_embedding)
